# baseline (speedup 1.0000x reference)
import functools

import jax
import jax.numpy as jnp
from jax import lax
from jax.experimental import pallas as pl
from jax.experimental.pallas import tpu as pltpu
from jax.experimental.pallas import tpu_sc as plsc

_N_TAB = 49
_D = 128
_G = 128
_C = 256
_NBUF = 3
_PREF = 2
_NC = 2
_NS = 16
_NW = _NC * _NS


def _sc_body(nsteps, seq_hbm, idx_hbm, tab_hbm, out_hbm,
             tab_sh, idx_v, seq_v,
             ssems, gsems, osems):
    cid = lax.axis_index("c")
    sid = lax.axis_index("s")
    wid = sid * _NC + cid
    chunk0 = wid * nsteps

    @pl.when(sid == 0)
    def _():
        pltpu.sync_copy(tab_hbm, tab_sh)
    plsc.subcore_barrier()

    pltpu.sync_copy(idx_hbm.at[pl.ds(chunk0 * 2, nsteps * 2)], idx_v)

    def issue_in(g, b):
        pltpu.async_copy(seq_hbm.at[pl.ds((chunk0 + g) * _C, _C)],
                         seq_v.at[b], ssems.at[b])

    def prep_gather(g, b):
        pltpu.make_async_copy(seq_hbm.at[pl.ds(0, _C)], seq_v.at[b],
                              ssems.at[b]).wait()
        for j in range(2):
            for i in range(_G // 16):
                v = idx_v[2 * g + j, pl.ds(i * 16, 16)]
                idx_v[2 * g + j, pl.ds(i * 16, 16)] = jnp.where(
                    v == -1, _N_TAB - 1, v)
            pltpu.async_copy(tab_sh.at[idx_v.at[2 * g + j]],
                             seq_v.at[b, pl.ds(j * _G, _G)],
                             gsems.at[b], add=True)

    def wait_gather(g, b):
        for j in range(2):
            pltpu.make_async_copy(tab_sh.at[idx_v.at[2 * g + j]],
                                  seq_v.at[b, pl.ds(j * _G, _G)],
                                  gsems.at[b]).wait()

    def issue_out(g, b):
        pltpu.async_copy(seq_v.at[b],
                         out_hbm.at[pl.ds((chunk0 + g) * _C, _C)],
                         osems.at[b])

    def wait_out_slot(b):
        pltpu.make_async_copy(seq_v.at[b], out_hbm.at[pl.ds(0, _C)],
                              osems.at[b]).wait()

    for b in range(_PREF):
        issue_in(b, b)
    prep_gather(0, 0)

    def iteration(g, b):
        wait_gather(g, b)
        issue_out(g, b)
        nb1 = (b + 1) % _NBUF

        @pl.when(g + 1 < nsteps)
        def _():
            prep_gather(g + 1, nb1)

        nbr = (b + _PREF) % _NBUF

        @pl.when(g + _PREF < nsteps)
        def _():
            @pl.when(g >= _NBUF - _PREF)
            def _():
                wait_out_slot(nbr)
            issue_in(g + _PREF, nbr)

    def group(grp, carry):
        for b in range(_NBUF):
            iteration(grp * _NBUF + b, b)
        return carry

    ngroups = nsteps // _NBUF
    lax.fori_loop(0, ngroups, group, 0)
    for r in range(nsteps - ngroups * _NBUF):
        iteration(ngroups * _NBUF + r, r)

    for b in range(_NBUF):
        wait_out_slot(b)


def kernel(seq, time_stamp, time_embed):
    B, L, D = seq.shape
    n = B * L
    seq2 = seq.reshape(n, D)
    idx2 = time_stamp.reshape(-1).astype(jnp.int32).reshape(n // _G, _G)
    nsteps = n // (_NW * _C)
    mesh = plsc.VectorSubcoreMesh(core_axis_name="c", subcore_axis_name="s")
    out = pl.kernel(
        functools.partial(_sc_body, nsteps),
        out_type=jax.ShapeDtypeStruct((n, D), jnp.float32),
        mesh=mesh,
        scratch_types=[
            pltpu.VMEM_SHARED((_N_TAB, _D), jnp.float32),
            pltpu.VMEM((2 * n // (_NW * _C), _G), jnp.int32),
            pltpu.VMEM((_NBUF, _C, _D), jnp.float32),
            pltpu.SemaphoreType.DMA((_NBUF,)),
            pltpu.SemaphoreType.DMA((_NBUF,)),
            pltpu.SemaphoreType.DMA((_NBUF,)),
        ],
    )(seq2, idx2, time_embed)
    return out.reshape(B, L, D)
